# Initial kernel scaffold; baseline (speedup 1.0000x reference)
import jax
import jax.numpy as jnp
from jax.experimental import pallas as pl


def kernel(x, edge_index, batch, W11, b11, W12, b12, g1, be1, W21, b21, W22, b22, g2, be2, W31, b31, W32, b32, g3, be3, Wb, bb, Wm, bm):
    raise NotImplementedError("write your pallas kernel here")


# speedup vs baseline: 4.0512x; 4.0512x over previous
import jax
import jax.numpy as jnp
from jax import lax
from jax.experimental import pallas as pl
from jax.experimental.pallas import tpu as pltpu
from jax.experimental.pallas import tpu_sc as plsc

_N = 10000
_E = 320000
_G = 64
_D = 32
_DP = 128
_NP = 10112
_NTILES = 32
_CHUNK = 128
_CH = 79
_EPAD = _NTILES * _CH * _CHUNK


def _agg_body(y_hbm, src_hbm, dst_hbm, z_hbm, out_hbm, src_v, dst_v, rows_v,
              acc_sh, sem):
    c = lax.axis_index("c")
    s = lax.axis_index("s")
    wid = c * 16 + s

    @pl.when(s == 0)
    def _():
        pltpu.sync_copy(z_hbm, acc_sh)

    plsc.subcore_barrier()

    pltpu.sync_copy(src_hbm.at[wid], src_v)
    pltpu.sync_copy(dst_hbm.at[wid], dst_v)

    def chunk(j, carry):
        pltpu.async_copy(y_hbm.at[src_v.at[j]], rows_v, sem).wait()
        pltpu.sync_copy(rows_v, acc_sh.at[dst_v.at[j]], add=True)
        return carry

    lax.fori_loop(0, _CH, chunk, 0)
    plsc.subcore_barrier()

    rows = _NP // 16
    pltpu.sync_copy(acc_sh.at[pl.ds(s * rows, rows)],
                    out_hbm.at[c, pl.ds(s * rows, rows)])


def _agg_sc(y, srcr, dstr, zeros):
    return pl.kernel(
        _agg_body,
        out_type=jax.ShapeDtypeStruct((2, _NP, _DP), jnp.float32),
        mesh=plsc.VectorSubcoreMesh(core_axis_name="c", subcore_axis_name="s"),
        scratch_types=[
            pltpu.VMEM((_CH, _CHUNK), jnp.int32),
            pltpu.VMEM((_CH, _CHUNK), jnp.int32),
            pltpu.VMEM((_CHUNK, _DP), jnp.float32),
            pltpu.VMEM_SHARED((_NP, _DP), jnp.float32),
            pltpu.SemaphoreType.DMA,
        ],
    )(y, srcr, dstr, zeros)


def _bn_tail(z, g_ref, be_ref):
    ridx = lax.broadcasted_iota(jnp.int32, z.shape, 0)
    z = jnp.where(ridx < _N, z, 0.0)
    mu = jnp.sum(z, axis=0, keepdims=True) * (1.0 / _N)
    ex2 = jnp.sum(z * z, axis=0, keepdims=True) * (1.0 / _N)
    var = ex2 - mu * mu
    return (z - mu) * lax.rsqrt(var + 1e-5) * g_ref[...] + be_ref[...]


def _conv_chain(x_ref, p_ref, w1_ref, b1_ref, w2_ref, b2_ref):
    t = x_ref[...] + p_ref[0] + p_ref[1]
    h = jnp.dot(t, w1_ref[...], preferred_element_type=jnp.float32) + b1_ref[...]
    h = jnp.maximum(h, 0.0)
    z = jnp.dot(h, w2_ref[...], preferred_element_type=jnp.float32) + b2_ref[...]
    return jnp.maximum(z, 0.0)


def _stage_body(x_ref, p_ref, w1_ref, b1_ref, w2_ref, b2_ref, g_ref, be_ref,
                o_ref):
    z = _conv_chain(x_ref, p_ref, w1_ref, b1_ref, w2_ref, b2_ref)
    o_ref[...] = _bn_tail(z, g_ref, be_ref)


def _stage(x, p, w1, b1, w2, b2, g, be):
    return pl.pallas_call(
        _stage_body,
        out_shape=jax.ShapeDtypeStruct((_NP, _DP), jnp.float32),
    )(x, p, w1, b1, w2, b2, g, be)


def _final_body(x_ref, p_ref, w1_ref, b1_ref, w2_ref, b2_ref, g_ref, be_ref,
                batch_ref, wb_ref, bb_ref, wm_ref, bm_ref, o_ref):
    z = _conv_chain(x_ref, p_ref, w1_ref, b1_ref, w2_ref, b2_ref)
    x3 = _bn_tail(z, g_ref, be_ref)
    oh = (batch_ref[...] == lax.broadcasted_iota(jnp.int32, (_NP, _G), 1))
    oh = oh.astype(jnp.float32)
    cnt = jnp.sum(oh, axis=0)
    sums = lax.dot_general(oh, x3, (((0,), (0,)), ((), ())),
                           preferred_element_type=jnp.float32)
    pooled = sums / jnp.maximum(cnt, 1.0)[:, None]
    hh = jnp.dot(pooled, wb_ref[...], preferred_element_type=jnp.float32)
    hh = jnp.maximum(hh + bb_ref[...], 0.0)
    logits = jnp.dot(hh, wm_ref[...],
                     preferred_element_type=jnp.float32) + bm_ref[...]
    o_ref[...] = jax.nn.sigmoid(logits)


def _final(x, p, w1, b1, w2, b2, g, be, batch_p, wb, bb, wm, bm):
    return pl.pallas_call(
        _final_body,
        out_shape=jax.ShapeDtypeStruct((_G, 1), jnp.float32),
    )(x, p, w1, b1, w2, b2, g, be, batch_p, wb, bb, wm, bm)


def kernel(x, edge_index, batch, W11, b11, W12, b12, g1, be1, W21, b21, W22,
           b22, g2, be2, W31, b31, W32, b32, g3, be3, Wb, bb, Wm, bm):
    src = edge_index[0].astype(jnp.int32)
    dst = edge_index[1].astype(jnp.int32)
    pad = _EPAD - _E
    srcr = jnp.concatenate([src, jnp.full((pad,), _N, jnp.int32)])
    dstr = jnp.concatenate([dst, jnp.full((pad,), _N, jnp.int32)])
    srcr = srcr.reshape(_NTILES, _CH, _CHUNK)
    dstr = dstr.reshape(_NTILES, _CH, _CHUNK)

    zeros = jnp.zeros((_NP, _DP), jnp.float32)
    x_pad = jnp.pad(x, ((0, _NP - _N), (0, 0)))
    batch_p = jnp.pad(batch.astype(jnp.int32), (0, _NP - _N),
                      constant_values=_G).reshape(_NP, 1)

    r = lambda v: v.reshape(1, -1)
    padc = lambda w: jnp.pad(w, ((0, 0), (0, _DP - _D)))
    padr = lambda w: jnp.pad(w, ((0, _DP - _D), (0, 0)))
    padv = lambda v: jnp.pad(v, (0, _DP - _D)).reshape(1, -1)

    p = _agg_sc(x_pad, srcr, dstr, zeros)
    x1 = _stage(x_pad, p, W11, r(b11), padc(W12), padv(b12), padv(g1),
                padv(be1))
    p = _agg_sc(x1, srcr, dstr, zeros)
    x2 = _stage(x1, p, padr(W21), r(b21), padc(W22), padv(b22), padv(g2),
                padv(be2))
    p = _agg_sc(x2, srcr, dstr, zeros)
    return _final(x2, p, padr(W31), r(b31), W32, r(b32), r(g3), r(be3),
                  batch_p, Wb, r(bb), Wm, r(bm))
